# Initial kernel scaffold; baseline (speedup 1.0000x reference)
import jax
import jax.numpy as jnp
from jax.experimental import pallas as pl


def kernel(xyz1, xyz2, points1, points2, W0, b0, g0, beta0, W1, b1, g1, beta1, W2, b2, g2, beta2):
    raise NotImplementedError("write your pallas kernel here")


# speedup vs baseline: 11.5576x; 11.5576x over previous
import functools

import jax
import jax.numpy as jnp
from jax.experimental import pallas as pl
from jax.experimental.pallas import tpu as pltpu

B, N, S = 8, 4096, 1024
C1, C2 = 256, 512
TN = 512
NT = N // TN
NSTEPS = B * NT
NPTS = float(B * N)
HIGH = jax.lax.Precision.HIGHEST


def _interp_l0_body(xyz1_ref, xyz2t_ref, p1_ref, p2_ref, w0a_ref, w0b_ref,
                    b0_ref, y_ref, st_ref, acc1, acc2):
    b = pl.program_id(0)
    nt = pl.program_id(1)
    step = b * NT + nt

    x1 = xyz1_ref[0]
    z2 = xyz2t_ref[0]
    s_src = jnp.sum(x1 * x1, axis=0, keepdims=True)
    d_dst = (z2[:, 0:1] * z2[:, 0:1] + z2[:, 1:2] * z2[:, 1:2]
             + z2[:, 2:3] * z2[:, 2:3])
    cross = jnp.dot(z2, x1, preferred_element_type=jnp.float32)
    dist = d_dst + s_src - 2.0 * cross

    iota = jax.lax.broadcasted_iota(jnp.int32, (S, TN), 0)
    recips = []
    sels = []
    for _ in range(3):
        mval = jnp.min(dist, axis=0, keepdims=True)
        eq = dist == mval
        idx = jnp.min(jnp.where(eq, iota, jnp.int32(S)), axis=0,
                      keepdims=True)
        sel = iota == idx
        recips.append(1.0 / (mval + 1e-8))
        sels.append(sel)
        dist = jnp.where(sel, jnp.float32(jnp.inf), dist)
    norm = recips[0] + recips[1] + recips[2]
    aw = ((recips[0] / norm) * sels[0].astype(jnp.float32)
          + (recips[1] / norm) * sels[1].astype(jnp.float32)
          + (recips[2] / norm) * sels[2].astype(jnp.float32))

    interp = jnp.dot(p2_ref[0], aw, preferred_element_type=jnp.float32,
                     precision=HIGH)
    y = (jnp.dot(w0a_ref[...], p1_ref[0], preferred_element_type=jnp.float32)
         + jnp.dot(w0b_ref[...], interp, preferred_element_type=jnp.float32)
         + b0_ref[...])
    y_ref[0] = y

    @pl.when(step == 0)
    def _():
        acc1[...] = y
        acc2[...] = y * y

    @pl.when(step != 0)
    def _():
        acc1[...] += y
        acc2[...] += y * y

    @pl.when(step == NSTEPS - 1)
    def _():
        st_ref[:, 0:1] = jnp.sum(acc1[...], axis=1, keepdims=True)
        st_ref[:, 1:2] = jnp.sum(acc2[...], axis=1, keepdims=True)


def _bn_relu(st_ref, g_ref, beta_ref):
    mean = st_ref[:, 0:1] * (1.0 / NPTS)
    var = st_ref[:, 1:2] * (1.0 / NPTS) - mean * mean
    a = g_ref[...] * jax.lax.rsqrt(var + 1e-5)
    c = beta_ref[...] - a * mean
    return a, c


def _layer_body(y_ref, st_ref, g_ref, beta_ref, w_ref, b_ref,
                yo_ref, sto_ref, acc1, acc2):
    b = pl.program_id(0)
    nt = pl.program_id(1)
    step = b * NT + nt
    a, c = _bn_relu(st_ref, g_ref, beta_ref)
    x = jnp.maximum(a * y_ref[0] + c, 0.0)
    y = jnp.dot(w_ref[...], x, preferred_element_type=jnp.float32) + b_ref[...]
    yo_ref[0] = y

    @pl.when(step == 0)
    def _():
        acc1[...] = y
        acc2[...] = y * y

    @pl.when(step != 0)
    def _():
        acc1[...] += y
        acc2[...] += y * y

    @pl.when(step == NSTEPS - 1)
    def _():
        sto_ref[:, 0:1] = jnp.sum(acc1[...], axis=1, keepdims=True)
        sto_ref[:, 1:2] = jnp.sum(acc2[...], axis=1, keepdims=True)


def _final_body(y_ref, st_ref, g_ref, beta_ref, o_ref):
    a, c = _bn_relu(st_ref, g_ref, beta_ref)
    o_ref[0] = jnp.maximum(a * y_ref[0] + c, 0.0)


def _col(v):
    return v.reshape(-1, 1)


def kernel(xyz1, xyz2, points1, points2, W0, b0, g0, beta0,
           W1, b1, g1, beta1, W2, b2, g2, beta2):
    xyz2t = jnp.transpose(xyz2, (0, 2, 1))
    w0a, w0b = W0[:, :C1], W0[:, C1:]

    full = lambda shp: pl.BlockSpec(shp, lambda b, n: (0, 0))
    tile = lambda c: pl.BlockSpec((1, c, TN), lambda b, n: (b, 0, n))
    perb = lambda r, c: pl.BlockSpec((1, r, c), lambda b, n: (b, 0, 0))

    y0, st0 = pl.pallas_call(
        _interp_l0_body,
        grid=(B, NT),
        in_specs=[tile(3), perb(S, 3), tile(C1), perb(C2, S),
                  full((512, C1)), full((512, C2)), full((512, 1))],
        out_specs=[tile(512), full((512, 2))],
        out_shape=[jax.ShapeDtypeStruct((B, 512, N), jnp.float32),
                   jax.ShapeDtypeStruct((512, 2), jnp.float32)],
        scratch_shapes=[pltpu.VMEM((512, TN), jnp.float32),
                        pltpu.VMEM((512, TN), jnp.float32)],
    )(xyz1, xyz2t, points1, points2, w0a, w0b, _col(b0))

    def layer(y, st, g, beta, W, bias, c_in, c_out):
        return pl.pallas_call(
            functools.partial(_layer_body),
            grid=(B, NT),
            in_specs=[tile(c_in), full((c_in, 2)), full((c_in, 1)),
                      full((c_in, 1)), full((c_out, c_in)), full((c_out, 1))],
            out_specs=[tile(c_out), full((c_out, 2))],
            out_shape=[jax.ShapeDtypeStruct((B, c_out, N), jnp.float32),
                       jax.ShapeDtypeStruct((c_out, 2), jnp.float32)],
            scratch_shapes=[pltpu.VMEM((c_out, TN), jnp.float32),
                            pltpu.VMEM((c_out, TN), jnp.float32)],
        )(y, st, _col(g), _col(beta), W, _col(bias))

    y1, st1 = layer(y0, st0, g0, beta0, W1, b1, 512, 512)
    y2, st2 = layer(y1, st1, g1, beta1, W2, b2, 512, 256)

    out = pl.pallas_call(
        _final_body,
        grid=(B, NT),
        in_specs=[tile(256), full((256, 2)), full((256, 1)), full((256, 1))],
        out_specs=tile(256),
        out_shape=jax.ShapeDtypeStruct((B, 256, N), jnp.float32),
    )(y2, st2, _col(g2), _col(beta2))
    return out


# speedup vs baseline: 13.2250x; 1.1443x over previous
import functools

import jax
import jax.numpy as jnp
from jax.experimental import pallas as pl
from jax.experimental.pallas import tpu as pltpu

B, N, S = 8, 4096, 1024
C1, C2 = 256, 512
TN = 512
NT = N // TN
NSTEPS = B * NT
NPTS = float(B * N)
HIGH = jax.lax.Precision.HIGHEST


def _interp_l0_body(xyz1_ref, xyz2t_ref, p1_ref, p2_ref, w0a_ref, w0b_ref,
                    b0_ref, y_ref, st_ref, acc1, acc2):
    b = pl.program_id(0)
    nt = pl.program_id(1)
    step = b * NT + nt

    x1 = xyz1_ref[0]
    z2 = xyz2t_ref[0]
    s_src = jnp.sum(x1 * x1, axis=0, keepdims=True)
    d_dst = (z2[:, 0:1] * z2[:, 0:1] + z2[:, 1:2] * z2[:, 1:2]
             + z2[:, 2:3] * z2[:, 2:3])
    cross = jnp.dot(z2, x1, preferred_element_type=jnp.float32)
    dist = d_dst + s_src - 2.0 * cross

    iota = jax.lax.broadcasted_iota(jnp.int32, (S, TN), 0)
    recips = []
    sels = []
    for _ in range(3):
        mval = jnp.min(dist, axis=0, keepdims=True)
        eq = dist == mval
        idx = jnp.min(jnp.where(eq, iota, jnp.int32(S)), axis=0,
                      keepdims=True)
        sel = iota == idx
        recips.append(1.0 / (mval + 1e-8))
        sels.append(sel)
        dist = jnp.where(sel, jnp.float32(jnp.inf), dist)
    norm = recips[0] + recips[1] + recips[2]
    aw = ((recips[0] / norm) * sels[0].astype(jnp.float32)
          + (recips[1] / norm) * sels[1].astype(jnp.float32)
          + (recips[2] / norm) * sels[2].astype(jnp.float32))

    p2 = p2_ref[0]
    p2h = p2.astype(jnp.bfloat16)
    p2l = (p2 - p2h.astype(jnp.float32)).astype(jnp.bfloat16)
    awh = aw.astype(jnp.bfloat16)
    awl = (aw - awh.astype(jnp.float32)).astype(jnp.bfloat16)
    interp = (jnp.dot(p2h, awh, preferred_element_type=jnp.float32)
              + jnp.dot(p2h, awl, preferred_element_type=jnp.float32)
              + jnp.dot(p2l, awh, preferred_element_type=jnp.float32))
    y = (jnp.dot(w0a_ref[...], p1_ref[0], preferred_element_type=jnp.float32)
         + jnp.dot(w0b_ref[...], interp, preferred_element_type=jnp.float32)
         + b0_ref[...])
    y_ref[0] = y

    @pl.when(step == 0)
    def _():
        acc1[...] = y
        acc2[...] = y * y

    @pl.when(step != 0)
    def _():
        acc1[...] += y
        acc2[...] += y * y

    @pl.when(step == NSTEPS - 1)
    def _():
        st_ref[:, 0:1] = jnp.sum(acc1[...], axis=1, keepdims=True)
        st_ref[:, 1:2] = jnp.sum(acc2[...], axis=1, keepdims=True)


def _bn_relu(st_ref, g_ref, beta_ref):
    mean = st_ref[:, 0:1] * (1.0 / NPTS)
    var = st_ref[:, 1:2] * (1.0 / NPTS) - mean * mean
    a = g_ref[...] * jax.lax.rsqrt(var + 1e-5)
    c = beta_ref[...] - a * mean
    return a, c


def _layer_body(y_ref, st_ref, g_ref, beta_ref, w_ref, b_ref,
                yo_ref, sto_ref, acc1, acc2):
    b = pl.program_id(0)
    nt = pl.program_id(1)
    step = b * NT + nt
    a, c = _bn_relu(st_ref, g_ref, beta_ref)
    x = jnp.maximum(a * y_ref[0] + c, 0.0)
    y = jnp.dot(w_ref[...], x, preferred_element_type=jnp.float32) + b_ref[...]
    yo_ref[0] = y

    @pl.when(step == 0)
    def _():
        acc1[...] = y
        acc2[...] = y * y

    @pl.when(step != 0)
    def _():
        acc1[...] += y
        acc2[...] += y * y

    @pl.when(step == NSTEPS - 1)
    def _():
        sto_ref[:, 0:1] = jnp.sum(acc1[...], axis=1, keepdims=True)
        sto_ref[:, 1:2] = jnp.sum(acc2[...], axis=1, keepdims=True)


def _final_body(y_ref, st_ref, g_ref, beta_ref, o_ref):
    a, c = _bn_relu(st_ref, g_ref, beta_ref)
    o_ref[0] = jnp.maximum(a * y_ref[0] + c, 0.0)


def _col(v):
    return v.reshape(-1, 1)


def kernel(xyz1, xyz2, points1, points2, W0, b0, g0, beta0,
           W1, b1, g1, beta1, W2, b2, g2, beta2):
    xyz2t = jnp.transpose(xyz2, (0, 2, 1))
    w0a, w0b = W0[:, :C1], W0[:, C1:]

    full = lambda shp: pl.BlockSpec(shp, lambda b, n: (0, 0))
    tile = lambda c: pl.BlockSpec((1, c, TN), lambda b, n: (b, 0, n))
    perb = lambda r, c: pl.BlockSpec((1, r, c), lambda b, n: (b, 0, 0))

    y0, st0 = pl.pallas_call(
        _interp_l0_body,
        grid=(B, NT),
        in_specs=[tile(3), perb(S, 3), tile(C1), perb(C2, S),
                  full((512, C1)), full((512, C2)), full((512, 1))],
        out_specs=[tile(512), full((512, 2))],
        out_shape=[jax.ShapeDtypeStruct((B, 512, N), jnp.float32),
                   jax.ShapeDtypeStruct((512, 2), jnp.float32)],
        scratch_shapes=[pltpu.VMEM((512, TN), jnp.float32),
                        pltpu.VMEM((512, TN), jnp.float32)],
    )(xyz1, xyz2t, points1, points2, w0a, w0b, _col(b0))

    def layer(y, st, g, beta, W, bias, c_in, c_out):
        return pl.pallas_call(
            functools.partial(_layer_body),
            grid=(B, NT),
            in_specs=[tile(c_in), full((c_in, 2)), full((c_in, 1)),
                      full((c_in, 1)), full((c_out, c_in)), full((c_out, 1))],
            out_specs=[tile(c_out), full((c_out, 2))],
            out_shape=[jax.ShapeDtypeStruct((B, c_out, N), jnp.float32),
                       jax.ShapeDtypeStruct((c_out, 2), jnp.float32)],
            scratch_shapes=[pltpu.VMEM((c_out, TN), jnp.float32),
                            pltpu.VMEM((c_out, TN), jnp.float32)],
        )(y, st, _col(g), _col(beta), W, _col(bias))

    y1, st1 = layer(y0, st0, g0, beta0, W1, b1, 512, 512)
    y2, st2 = layer(y1, st1, g1, beta1, W2, b2, 512, 256)

    out = pl.pallas_call(
        _final_body,
        grid=(B, NT),
        in_specs=[tile(256), full((256, 2)), full((256, 1)), full((256, 1))],
        out_specs=tile(256),
        out_shape=jax.ShapeDtypeStruct((B, 256, N), jnp.float32),
    )(y2, st2, _col(g2), _col(beta2))
    return out


# speedup vs baseline: 13.2326x; 1.0006x over previous
import functools

import jax
import jax.numpy as jnp
from jax.experimental import pallas as pl
from jax.experimental.pallas import tpu as pltpu

B, N, S = 8, 4096, 1024
C1, C2 = 256, 512
TN = 512
NT = N // TN
NSTEPS = B * NT
NPTS = float(B * N)
HIGH = jax.lax.Precision.HIGHEST


def _interp_l0_body(xyz1_ref, xyz2t_ref, p1_ref, p2_ref, w0a_ref, w0b_ref,
                    b0_ref, y_ref, st_ref, acc1, acc2):
    b = pl.program_id(0)
    nt = pl.program_id(1)
    step = b * NT + nt

    x1 = xyz1_ref[0]
    z2 = xyz2t_ref[0]
    s_src = (x1[0:1, :] * x1[0:1, :] + x1[1:2, :] * x1[1:2, :]
             + x1[2:3, :] * x1[2:3, :])
    d_dst = (z2[:, 0:1] * z2[:, 0:1] + z2[:, 1:2] * z2[:, 1:2]
             + z2[:, 2:3] * z2[:, 2:3])
    cross = jnp.dot(z2, x1, preferred_element_type=jnp.float32)
    dist = d_dst + s_src - 2.0 * cross

    iota = jax.lax.broadcasted_iota(jnp.int32, (S, TN), 0)
    recips = []
    sels = []
    for _ in range(3):
        mval = jnp.min(dist, axis=0, keepdims=True)
        eq = dist == mval
        idx = jnp.min(jnp.where(eq, iota, jnp.int32(S)), axis=0,
                      keepdims=True)
        sel = iota == idx
        recips.append(1.0 / (mval + 1e-8))
        sels.append(sel)
        dist = jnp.where(sel, jnp.float32(jnp.inf), dist)
    norm = recips[0] + recips[1] + recips[2]
    aw = ((recips[0] / norm) * sels[0].astype(jnp.float32)
          + (recips[1] / norm) * sels[1].astype(jnp.float32)
          + (recips[2] / norm) * sels[2].astype(jnp.float32))

    p2 = p2_ref[0]
    p2h = p2.astype(jnp.bfloat16)
    p2l = (p2 - p2h.astype(jnp.float32)).astype(jnp.bfloat16)
    awh = aw.astype(jnp.bfloat16)
    awl = (aw - awh.astype(jnp.float32)).astype(jnp.bfloat16)
    interp = (jnp.dot(p2h, awh, preferred_element_type=jnp.float32)
              + jnp.dot(p2h, awl, preferred_element_type=jnp.float32)
              + jnp.dot(p2l, awh, preferred_element_type=jnp.float32))
    y = (jnp.dot(w0a_ref[...], p1_ref[0], preferred_element_type=jnp.float32)
         + jnp.dot(w0b_ref[...], interp, preferred_element_type=jnp.float32)
         + b0_ref[...])
    y_ref[0] = y

    @pl.when(step == 0)
    def _():
        acc1[...] = y
        acc2[...] = y * y

    @pl.when(step != 0)
    def _():
        acc1[...] += y
        acc2[...] += y * y

    @pl.when(step == NSTEPS - 1)
    def _():
        st_ref[:, 0:1] = jnp.sum(acc1[...], axis=1, keepdims=True)
        st_ref[:, 1:2] = jnp.sum(acc2[...], axis=1, keepdims=True)


def _bn_relu(st_ref, g_ref, beta_ref):
    mean = st_ref[:, 0:1] * (1.0 / NPTS)
    var = st_ref[:, 1:2] * (1.0 / NPTS) - mean * mean
    a = g_ref[...] * jax.lax.rsqrt(var + 1e-5)
    c = beta_ref[...] - a * mean
    return a, c


def _layer_body(y_ref, st_ref, g_ref, beta_ref, w_ref, b_ref,
                yo_ref, sto_ref, acc1, acc2):
    b = pl.program_id(0)
    nt = pl.program_id(1)
    step = b * NT + nt
    a, c = _bn_relu(st_ref, g_ref, beta_ref)
    x = jnp.maximum(a * y_ref[0] + c, 0.0)
    y = jnp.dot(w_ref[...], x, preferred_element_type=jnp.float32) + b_ref[...]
    yo_ref[0] = y

    @pl.when(step == 0)
    def _():
        acc1[...] = y
        acc2[...] = y * y

    @pl.when(step != 0)
    def _():
        acc1[...] += y
        acc2[...] += y * y

    @pl.when(step == NSTEPS - 1)
    def _():
        sto_ref[:, 0:1] = jnp.sum(acc1[...], axis=1, keepdims=True)
        sto_ref[:, 1:2] = jnp.sum(acc2[...], axis=1, keepdims=True)


def _final_body(y_ref, st_ref, g_ref, beta_ref, o_ref):
    a, c = _bn_relu(st_ref, g_ref, beta_ref)
    o_ref[0] = jnp.maximum(a * y_ref[0] + c, 0.0)


def _col(v):
    return v.reshape(-1, 1)


def kernel(xyz1, xyz2, points1, points2, W0, b0, g0, beta0,
           W1, b1, g1, beta1, W2, b2, g2, beta2):
    xyz2t = jnp.transpose(xyz2, (0, 2, 1))
    w0a, w0b = W0[:, :C1], W0[:, C1:]

    full = lambda shp: pl.BlockSpec(shp, lambda b, n: (0, 0))
    tile = lambda c: pl.BlockSpec((1, c, TN), lambda b, n: (b, 0, n))
    perb = lambda r, c: pl.BlockSpec((1, r, c), lambda b, n: (b, 0, 0))

    y0, st0 = pl.pallas_call(
        _interp_l0_body,
        grid=(B, NT),
        in_specs=[tile(3), perb(S, 3), tile(C1), perb(C2, S),
                  full((512, C1)), full((512, C2)), full((512, 1))],
        out_specs=[tile(512), full((512, 2))],
        out_shape=[jax.ShapeDtypeStruct((B, 512, N), jnp.float32),
                   jax.ShapeDtypeStruct((512, 2), jnp.float32)],
        scratch_shapes=[pltpu.VMEM((512, TN), jnp.float32),
                        pltpu.VMEM((512, TN), jnp.float32)],
    )(xyz1, xyz2t, points1, points2, w0a, w0b, _col(b0))

    def layer(y, st, g, beta, W, bias, c_in, c_out):
        return pl.pallas_call(
            functools.partial(_layer_body),
            grid=(B, NT),
            in_specs=[tile(c_in), full((c_in, 2)), full((c_in, 1)),
                      full((c_in, 1)), full((c_out, c_in)), full((c_out, 1))],
            out_specs=[tile(c_out), full((c_out, 2))],
            out_shape=[jax.ShapeDtypeStruct((B, c_out, N), jnp.float32),
                       jax.ShapeDtypeStruct((c_out, 2), jnp.float32)],
            scratch_shapes=[pltpu.VMEM((c_out, TN), jnp.float32),
                            pltpu.VMEM((c_out, TN), jnp.float32)],
        )(y, st, _col(g), _col(beta), W, _col(bias))

    y1, st1 = layer(y0, st0, g0, beta0, W1, b1, 512, 512)
    y2, st2 = layer(y1, st1, g1, beta1, W2, b2, 512, 256)

    out = pl.pallas_call(
        _final_body,
        grid=(B, NT),
        in_specs=[tile(256), full((256, 2)), full((256, 1)), full((256, 1))],
        out_specs=tile(256),
        out_shape=jax.ShapeDtypeStruct((B, 256, N), jnp.float32),
    )(y2, st2, _col(g2), _col(beta2))
    return out
